# Initial kernel scaffold; baseline (speedup 1.0000x reference)
import jax
import jax.numpy as jnp
from jax.experimental import pallas as pl


def kernel(x, edge_index, W1, W2, fcW, fcb):
    raise NotImplementedError("write your pallas kernel here")


# speedup vs baseline: 29.1254x; 29.1254x over previous
import functools

import jax
import jax.numpy as jnp
from jax import lax
from jax.experimental import pallas as pl
from jax.experimental.pallas import tpu as pltpu
from jax.experimental.pallas import tpu_sc as plsc

N = 10000
D = 128
D_OUT = 64
E = 320000

NC = 2
NS = 16
NW = NC * NS
CHUNK = 128
NCHUNK = 80
NBUF = 2
GROUPS = NCHUNK // NBUF
EPT = CHUNK * NCHUNK
EPAD = NW * EPT
NROWS = 10112
RPT = NROWS // NS
NDEG = 12800
RPTD = NDEG // NS

@functools.cache
def _sc_kernels():
    mesh = plsc.VectorSubcoreMesh(core_axis_name="c", subcore_axis_name="s",
                                  num_cores=NC, num_subcores=NS)
    sc_deg = functools.partial(
        pl.kernel,
        out_type=jax.ShapeDtypeStruct((NC * NDEG,), jnp.float32),
        mesh=mesh,
        scratch_types=[
            pltpu.VMEM((NCHUNK, CHUNK), jnp.int32),
            pltpu.VMEM((CHUNK,), jnp.float32),
            pltpu.VMEM((CHUNK,), jnp.float32),
            pltpu.VMEM((RPTD,), jnp.float32),
            pltpu.MemorySpace.VMEM_SHARED((NDEG,), jnp.float32),
        ],
    )(_sc_deg_body)
    sc_layer = functools.partial(
        pl.kernel,
        out_type=jax.ShapeDtypeStruct((NC, NROWS, D), jnp.float32),
        mesh=mesh,
        scratch_types=[
            pltpu.VMEM((NCHUNK, CHUNK), jnp.int32),
            pltpu.VMEM((2, NBUF, CHUNK), jnp.int32),
            pltpu.VMEM((NBUF, CHUNK, D), jnp.float32),
            pltpu.MemorySpace.VMEM_SHARED((NROWS, D), jnp.float32),
        ] + [pltpu.SemaphoreType.DMA] * (2 * NBUF + 2),
    )(_sc_layer_body)
    return sc_deg, sc_layer


def _sc_deg_body(src_hbm, out_hbm, sidx, ones_v, zero_v, stage_v, acc):
    c = lax.axis_index("c")
    s = lax.axis_index("s")
    wid = c * NS + s
    for j in range(CHUNK // 16):
        ones_v[pl.ds(j * 16, 16)] = jnp.ones((16,), jnp.float32)
        zero_v[pl.ds(j * 16, 16)] = jnp.zeros((16,), jnp.float32)
    for j in range(RPTD // CHUNK):
        pltpu.sync_copy(zero_v, acc.at[pl.ds(s * RPTD + j * CHUNK, CHUNK)])
    rem = RPTD % CHUNK
    if rem:
        pltpu.sync_copy(zero_v.at[pl.ds(0, rem)],
                        acc.at[pl.ds(s * RPTD + (RPTD // CHUNK) * CHUNK, rem)])
    pltpu.sync_copy(src_hbm.at[pl.ds(wid * NCHUNK, NCHUNK)], sidx)
    plsc.subcore_barrier()

    def body(k, carry):
        pltpu.sync_copy(ones_v, acc.at[sidx.at[k]], add=True)
        return carry

    lax.fori_loop(0, NCHUNK, body, 0)
    plsc.subcore_barrier()
    pltpu.sync_copy(acc.at[pl.ds(s * RPTD, RPTD)], stage_v)
    pltpu.sync_copy(stage_v, out_hbm.at[pl.ds(c * NDEG + s * RPTD, RPTD)])


def _sc_layer_body(table_hbm, src_hbm, dst_hbm, zeros_hbm, out_hbm,
                   didx, sidx, rows, acc, *sems):
    c = lax.axis_index("c")
    s = lax.axis_index("s")
    wid = c * NS + s
    gsem = sems[:2 * NBUF]
    ssem = sems[2 * NBUF:]

    def gather_start(k_row, b):
        for t in range(2):
            pltpu.async_copy(
                table_hbm.at[didx.at[k_row, pl.ds(t * 64, 64)]],
                rows.at[b, pl.ds(t * 64, 64)], gsem[2 * b + t])

    def gather_wait(k_row, b):
        for t in range(2):
            pltpu.make_async_copy(
                table_hbm.at[didx.at[k_row, pl.ds(t * 64, 64)]],
                rows.at[b, pl.ds(t * 64, 64)], gsem[2 * b + t]).wait()
    pltpu.sync_copy(zeros_hbm.at[pl.ds(s * RPT, RPT)], acc.at[pl.ds(s * RPT, RPT)])
    pltpu.sync_copy(dst_hbm.at[pl.ds(wid * NCHUNK, NCHUNK)], didx)
    plsc.subcore_barrier()

    def sidx_src(g):
        return src_hbm.at[pl.ds(wid * NCHUNK + g * NBUF, NBUF)]

    for t in range(2):
        pltpu.async_copy(sidx_src(t), sidx.at[t], ssem[t])
    for b in range(NBUF):
        gather_start(b, b)

    def one_group(g, slot):
        k0 = g * NBUF
        pltpu.make_async_copy(sidx_src(g), sidx.at[slot], ssem[slot]).wait()
        for b in range(NBUF):
            k = k0 + b
            gather_wait(0, b)
            pltpu.sync_copy(rows.at[b], acc.at[sidx.at[slot, b]], add=True)

            @pl.when(g < GROUPS - 1)
            def _():
                gather_start(k + NBUF, b)

        @pl.when(g < GROUPS - 2)
        def _():
            pltpu.async_copy(sidx_src(g + 2), sidx.at[slot], ssem[slot])

    def super_group(h, carry):
        one_group(2 * h, 0)
        one_group(2 * h + 1, 1)
        return carry

    lax.fori_loop(0, GROUPS // 2, super_group, 0)
    plsc.subcore_barrier()
    pltpu.sync_copy(acc.at[pl.ds(s * RPT, RPT)], out_hbm.at[c, pl.ds(s * RPT, RPT)])


def _dinv_block(deg0_ref, deg1_ref):
    deg = 1.0 + deg0_ref[...] + deg1_ref[...]
    return lax.rsqrt(deg)


def _tc_a_body(x_ref, w1_ref, deg0_ref, deg1_ref, out_ref):
    p = lax.dot_general(x_ref[...], w1_ref[...], (((1,), (1,)), ((), ())),
                        preferred_element_type=jnp.float32)
    out_ref[...] = p * _dinv_block(deg0_ref, deg1_ref)


def _tc_b_body(zp_ref, yprev_ref, deg0_ref, deg1_ref, w2_ref, out_ref):
    dinv = _dinv_block(deg0_ref, deg1_ref)
    z = zp_ref[0] + zp_ref[1] + yprev_ref[...]
    h = jnp.maximum(dinv * z, 0.0)
    out_ref[...] = lax.dot_general(dinv * h, w2_ref[...], (((1,), (1,)), ((), ())),
                                   preferred_element_type=jnp.float32)


def _tc_c_body(zp_ref, yprev_ref, deg0_ref, deg1_ref, fcw_ref, fcb_ref, out_ref,
               acc_ref):
    i = pl.program_id(0)

    @pl.when(i == 0)
    def _():
        acc_ref[...] = jnp.zeros_like(acc_ref)

    dinv = _dinv_block(deg0_ref, deg1_ref)
    z = zp_ref[0] + zp_ref[1] + yprev_ref[...]
    h = jnp.maximum(dinv * z, 0.0)
    acc_ref[...] += jnp.sum(h, axis=0, keepdims=True)

    @pl.when(i == pl.num_programs(0) - 1)
    def _():
        m = acc_ref[...] * (1.0 / N)
        out_ref[...] = lax.dot_general(m, fcw_ref[...], (((1,), (1,)), ((), ())),
                                       preferred_element_type=jnp.float32) + fcb_ref[...]


_RB = 400


def _row_specs():
    deg_spec = pl.BlockSpec((_RB, 1), lambda i: (i, 0))
    row_spec = pl.BlockSpec((_RB, D), lambda i: (i, 0))
    z_spec = pl.BlockSpec((NC, _RB, D), lambda i: (0, i, 0))
    w_spec = pl.BlockSpec((D, D), lambda i: (0, 0))
    return deg_spec, row_spec, z_spec, w_spec


def _tc_a(x, w1, deg0, deg1):
    deg_spec, row_spec, _, w_spec = _row_specs()
    return pl.pallas_call(
        _tc_a_body,
        grid=(N // _RB,),
        in_specs=[row_spec, w_spec, deg_spec, deg_spec],
        out_specs=row_spec,
        out_shape=jax.ShapeDtypeStruct((N, D), jnp.float32),
    )(x, w1, deg0, deg1)


def _tc_b(zp, yprev, deg0, deg1, w2):
    deg_spec, row_spec, z_spec, w_spec = _row_specs()
    return pl.pallas_call(
        _tc_b_body,
        grid=(N // _RB,),
        in_specs=[z_spec, row_spec, deg_spec, deg_spec, w_spec],
        out_specs=row_spec,
        out_shape=jax.ShapeDtypeStruct((N, D), jnp.float32),
    )(zp, yprev, deg0, deg1, w2)


def _tc_c(zp, yprev, deg0, deg1, fcw, fcb):
    deg_spec, row_spec, z_spec, _ = _row_specs()
    fcw_spec = pl.BlockSpec((D_OUT * D_OUT, D), lambda i: (0, 0))
    fcb_spec = pl.BlockSpec((1, D_OUT * D_OUT), lambda i: (0, 0))
    out_spec = pl.BlockSpec((1, D_OUT * D_OUT), lambda i: (0, 0))
    return pl.pallas_call(
        _tc_c_body,
        grid=(N // _RB,),
        in_specs=[z_spec, row_spec, deg_spec, deg_spec, fcw_spec, fcb_spec],
        out_specs=out_spec,
        out_shape=jax.ShapeDtypeStruct((1, D_OUT * D_OUT), jnp.float32),
        scratch_shapes=[pltpu.VMEM((1, D), jnp.float32)],
    )(zp, yprev, deg0, deg1, fcw, fcb)


def kernel(x, edge_index, W1, W2, fcW, fcb):
    src = edge_index[0].astype(jnp.int32)
    dst = edge_index[1].astype(jnp.int32)
    npad = EPAD - E
    pad_i = jnp.arange(npad, dtype=jnp.int32)
    src_p = jnp.concatenate([src, N + pad_i % (NROWS - N)]).reshape(-1, CHUNK)
    dst_p = jnp.concatenate([dst, pad_i % N]).reshape(-1, CHUNK)

    zerosD = jnp.zeros((NROWS, D), jnp.float32)

    sc_deg, sc_layer = _sc_kernels()
    degp = sc_deg(src_p)
    deg0 = degp[0:N].reshape(N, 1)
    deg1 = degp[NDEG:NDEG + N].reshape(N, 1)
    y0 = _tc_a(x, W1, deg0, deg1)
    z1 = sc_layer(y0, src_p, dst_p, zerosD)
    y1 = _tc_b(z1, y0, deg0, deg1, W2)
    z2 = sc_layer(y1, src_p, dst_p, zerosD)
    a_flat = _tc_c(z2, y1, deg0, deg1, fcW, fcb.reshape(1, -1))
    return a_flat.reshape(D_OUT, D_OUT)


# speedup vs baseline: 32.3397x; 1.1104x over previous
import functools

import jax
import jax.numpy as jnp
from jax import lax
from jax.experimental import pallas as pl
from jax.experimental.pallas import tpu as pltpu
from jax.experimental.pallas import tpu_sc as plsc

N = 10000
D = 128
D_OUT = 64
E = 320000

NC = 2
NS = 16
NW = NC * NS
CHUNK = 128
NCHUNK = 80
NBUF = 2
GROUPS = NCHUNK // NBUF
EPT = CHUNK * NCHUNK
EPAD = NW * EPT
NROWS = 10112
RPT = NROWS // NS
NDEG = 12800
RPTD = NDEG // NS

@functools.cache
def _sc_kernels():
    mesh = plsc.VectorSubcoreMesh(core_axis_name="c", subcore_axis_name="s",
                                  num_cores=NC, num_subcores=NS)
    sc_deg = functools.partial(
        pl.kernel,
        out_type=jax.ShapeDtypeStruct((NC * NDEG,), jnp.float32),
        mesh=mesh,
        scratch_types=[
            pltpu.VMEM((NCHUNK, CHUNK), jnp.int32),
            pltpu.VMEM((CHUNK,), jnp.float32),
            pltpu.VMEM((CHUNK,), jnp.float32),
            pltpu.VMEM((RPTD,), jnp.float32),
            pltpu.MemorySpace.VMEM_SHARED((NDEG,), jnp.float32),
            pltpu.SemaphoreType.DMA,
        ],
    )(_sc_deg_body)
    sc_layer = functools.partial(
        pl.kernel,
        out_type=jax.ShapeDtypeStruct((NC, NROWS, D), jnp.float32),
        mesh=mesh,
        scratch_types=[
            pltpu.VMEM((NCHUNK, CHUNK), jnp.int32),
            pltpu.VMEM((2, NBUF, CHUNK), jnp.int32),
            pltpu.VMEM((NBUF, CHUNK, D), jnp.float32),
            pltpu.MemorySpace.VMEM_SHARED((NROWS, D), jnp.float32),
        ] + [pltpu.SemaphoreType.DMA] * (NBUF + 2),
    )(_sc_layer_body)
    return sc_deg, sc_layer


def _sc_deg_body(src_hbm, out_hbm, sidx, ones_v, zero_v, stage_v, acc, dsem):
    c = lax.axis_index("c")
    s = lax.axis_index("s")
    wid = c * NS + s
    for j in range(CHUNK // 16):
        ones_v[pl.ds(j * 16, 16)] = jnp.ones((16,), jnp.float32)
        zero_v[pl.ds(j * 16, 16)] = jnp.zeros((16,), jnp.float32)
    for j in range(RPTD // CHUNK):
        pltpu.sync_copy(zero_v, acc.at[pl.ds(s * RPTD + j * CHUNK, CHUNK)])
    rem = RPTD % CHUNK
    if rem:
        pltpu.sync_copy(zero_v.at[pl.ds(0, rem)],
                        acc.at[pl.ds(s * RPTD + (RPTD // CHUNK) * CHUNK, rem)])
    pltpu.sync_copy(src_hbm.at[pl.ds(wid * NCHUNK, NCHUNK)], sidx)
    plsc.subcore_barrier()

    LAG = 8

    def body(k, carry):
        pltpu.async_copy(ones_v, acc.at[sidx.at[k]], dsem, add=True)

        @pl.when(k >= LAG)
        def _():
            pltpu.make_async_copy(ones_v, acc.at[sidx.at[k - LAG]],
                                  dsem).wait()
        return carry

    lax.fori_loop(0, NCHUNK, body, 0)

    def drain(k, carry):
        pltpu.make_async_copy(ones_v, acc.at[sidx.at[k]], dsem).wait()
        return carry

    lax.fori_loop(NCHUNK - LAG, NCHUNK, drain, 0)
    plsc.subcore_barrier()
    pltpu.sync_copy(acc.at[pl.ds(s * RPTD, RPTD)], stage_v)
    pltpu.sync_copy(stage_v, out_hbm.at[pl.ds(c * NDEG + s * RPTD, RPTD)])


def _sc_layer_body(table_hbm, src_hbm, dst_hbm, zeros_hbm, out_hbm,
                   didx, sidx, rows, acc, *sems):
    c = lax.axis_index("c")
    s = lax.axis_index("s")
    wid = c * NS + s
    gsem = sems[:NBUF]
    ssem = sems[NBUF:]
    pltpu.sync_copy(zeros_hbm.at[pl.ds(s * RPT, RPT)], acc.at[pl.ds(s * RPT, RPT)])
    pltpu.sync_copy(dst_hbm.at[pl.ds(wid * NCHUNK, NCHUNK)], didx)
    plsc.subcore_barrier()

    def sidx_src(g):
        return src_hbm.at[pl.ds(wid * NCHUNK + g * NBUF, NBUF)]

    for t in range(2):
        pltpu.async_copy(sidx_src(t), sidx.at[t], ssem[t])
    for b in range(NBUF):
        pltpu.async_copy(table_hbm.at[didx.at[b]], rows.at[b], gsem[b])

    def one_group(g, slot):
        k0 = g * NBUF
        pltpu.make_async_copy(sidx_src(g), sidx.at[slot], ssem[slot]).wait()
        for b in range(NBUF):
            k = k0 + b
            pltpu.make_async_copy(table_hbm.at[didx.at[b]], rows.at[b],
                                  gsem[b]).wait()
            pltpu.sync_copy(rows.at[b], acc.at[sidx.at[slot, b]], add=True)

            @pl.when(g < GROUPS - 1)
            def _():
                pltpu.async_copy(table_hbm.at[didx.at[k + NBUF]], rows.at[b],
                                 gsem[b])

        @pl.when(g < GROUPS - 2)
        def _():
            pltpu.async_copy(sidx_src(g + 2), sidx.at[slot], ssem[slot])

    def super_group(h, carry):
        one_group(2 * h, 0)
        one_group(2 * h + 1, 1)
        return carry

    lax.fori_loop(0, GROUPS // 2, super_group, 0)
    plsc.subcore_barrier()
    pltpu.sync_copy(acc.at[pl.ds(s * RPT, RPT)], out_hbm.at[c, pl.ds(s * RPT, RPT)])


def _dinv_block(deg0_ref, deg1_ref):
    deg = 1.0 + deg0_ref[...] + deg1_ref[...]
    return lax.rsqrt(deg)


def _tc_a_body(x_ref, w1_ref, deg0_ref, deg1_ref, out_ref):
    p = lax.dot_general(x_ref[...], w1_ref[...], (((1,), (1,)), ((), ())),
                        preferred_element_type=jnp.float32)
    out_ref[...] = p * _dinv_block(deg0_ref, deg1_ref)


def _tc_b_body(zp_ref, yprev_ref, deg0_ref, deg1_ref, w2_ref, out_ref):
    dinv = _dinv_block(deg0_ref, deg1_ref)
    z = zp_ref[0] + zp_ref[1] + yprev_ref[...]
    h = jnp.maximum(dinv * z, 0.0)
    out_ref[...] = lax.dot_general(dinv * h, w2_ref[...], (((1,), (1,)), ((), ())),
                                   preferred_element_type=jnp.float32)


def _tc_c_body(zp_ref, yprev_ref, deg0_ref, deg1_ref, fcw_ref, fcb_ref, out_ref,
               acc_ref):
    i = pl.program_id(0)

    @pl.when(i == 0)
    def _():
        acc_ref[...] = jnp.zeros_like(acc_ref)

    dinv = _dinv_block(deg0_ref, deg1_ref)
    z = zp_ref[0] + zp_ref[1] + yprev_ref[...]
    h = jnp.maximum(dinv * z, 0.0)
    acc_ref[...] += jnp.sum(h, axis=0, keepdims=True)

    @pl.when(i == pl.num_programs(0) - 1)
    def _():
        m = acc_ref[...] * (1.0 / N)
        out_ref[...] = lax.dot_general(m, fcw_ref[...], (((1,), (1,)), ((), ())),
                                       preferred_element_type=jnp.float32) + fcb_ref[...]


_RB = 1000


def _row_specs():
    deg_spec = pl.BlockSpec((_RB, 1), lambda i: (i, 0))
    row_spec = pl.BlockSpec((_RB, D), lambda i: (i, 0))
    z_spec = pl.BlockSpec((NC, _RB, D), lambda i: (0, i, 0))
    w_spec = pl.BlockSpec((D, D), lambda i: (0, 0))
    return deg_spec, row_spec, z_spec, w_spec


def _tc_a(x, w1, deg0, deg1):
    deg_spec, row_spec, _, w_spec = _row_specs()
    return pl.pallas_call(
        _tc_a_body,
        grid=(N // _RB,),
        in_specs=[row_spec, w_spec, deg_spec, deg_spec],
        out_specs=row_spec,
        out_shape=jax.ShapeDtypeStruct((N, D), jnp.float32),
    )(x, w1, deg0, deg1)


def _tc_b(zp, yprev, deg0, deg1, w2):
    deg_spec, row_spec, z_spec, w_spec = _row_specs()
    return pl.pallas_call(
        _tc_b_body,
        grid=(N // _RB,),
        in_specs=[z_spec, row_spec, deg_spec, deg_spec, w_spec],
        out_specs=row_spec,
        out_shape=jax.ShapeDtypeStruct((N, D), jnp.float32),
    )(zp, yprev, deg0, deg1, w2)


def _tc_c(zp, yprev, deg0, deg1, fcw, fcb):
    deg_spec, row_spec, z_spec, _ = _row_specs()
    fcw_spec = pl.BlockSpec((D_OUT * D_OUT, D), lambda i: (0, 0))
    fcb_spec = pl.BlockSpec((1, D_OUT * D_OUT), lambda i: (0, 0))
    out_spec = pl.BlockSpec((1, D_OUT * D_OUT), lambda i: (0, 0))
    return pl.pallas_call(
        _tc_c_body,
        grid=(N // _RB,),
        in_specs=[z_spec, row_spec, deg_spec, deg_spec, fcw_spec, fcb_spec],
        out_specs=out_spec,
        out_shape=jax.ShapeDtypeStruct((1, D_OUT * D_OUT), jnp.float32),
        scratch_shapes=[pltpu.VMEM((1, D), jnp.float32)],
    )(zp, yprev, deg0, deg1, fcw, fcb)


def kernel(x, edge_index, W1, W2, fcW, fcb):
    src = edge_index[0].astype(jnp.int32)
    dst = edge_index[1].astype(jnp.int32)
    npad = EPAD - E
    pad_i = jnp.arange(npad, dtype=jnp.int32)
    src_p = jnp.concatenate([src, N + pad_i % (NROWS - N)]).reshape(-1, CHUNK)
    dst_p = jnp.concatenate([dst, pad_i % N]).reshape(-1, CHUNK)

    zerosD = jnp.zeros((NROWS, D), jnp.float32)

    sc_deg, sc_layer = _sc_kernels()
    degp = sc_deg(src_p)
    deg0 = degp[0:N].reshape(N, 1)
    deg1 = degp[NDEG:NDEG + N].reshape(N, 1)
    y0 = _tc_a(x, W1, deg0, deg1)
    z1 = sc_layer(y0, src_p, dst_p, zerosD)
    y1 = _tc_b(z1, y0, deg0, deg1, W2)
    z2 = sc_layer(y1, src_p, dst_p, zerosD)
    a_flat = _tc_c(z2, y1, deg0, deg1, fcW, fcb.reshape(1, -1))
    return a_flat.reshape(D_OUT, D_OUT)


# speedup vs baseline: 33.1999x; 1.0266x over previous
import functools

import jax
import jax.numpy as jnp
from jax import lax
from jax.experimental import pallas as pl
from jax.experimental.pallas import tpu as pltpu
from jax.experimental.pallas import tpu_sc as plsc

N = 10000
D = 128
D_OUT = 64
E = 320000

NC = 2
NS = 16
NW = NC * NS
CHUNK = 128
NCHUNK = 80
NBUF = 2
GROUPS = NCHUNK // NBUF
EPT = CHUNK * NCHUNK
EPAD = NW * EPT
NROWS = 10112
RPT = NROWS // NS
NDEG = 12800
RPTD = NDEG // NS

@functools.cache
def _sc_kernels():
    mesh = plsc.VectorSubcoreMesh(core_axis_name="c", subcore_axis_name="s",
                                  num_cores=NC, num_subcores=NS)
    sc_deg = functools.partial(
        pl.kernel,
        out_type=jax.ShapeDtypeStruct((NC * NDEG,), jnp.float32),
        mesh=mesh,
        scratch_types=[
            pltpu.VMEM((NCHUNK, CHUNK), jnp.int32),
            pltpu.VMEM((CHUNK,), jnp.float32),
            pltpu.VMEM((CHUNK,), jnp.float32),
            pltpu.VMEM((RPTD,), jnp.float32),
            pltpu.MemorySpace.VMEM_SHARED((NDEG,), jnp.float32),
            pltpu.SemaphoreType.DMA,
        ],
    )(_sc_deg_body)
    sc_layer = functools.partial(
        pl.kernel,
        out_type=jax.ShapeDtypeStruct((NC, NROWS, D), jnp.float32),
        mesh=mesh,
        scratch_types=[
            pltpu.VMEM((NCHUNK, CHUNK), jnp.int32),
            pltpu.VMEM((2, NBUF, CHUNK), jnp.int32),
            pltpu.VMEM((NBUF, CHUNK, D), jnp.float32),
            pltpu.MemorySpace.VMEM_SHARED((NROWS, D), jnp.float32),
        ] + [pltpu.SemaphoreType.DMA] * (NBUF + 3),
    )(_sc_layer_body)
    return sc_deg, sc_layer


def _sc_deg_body(src_hbm, out_hbm, sidx, ones_v, zero_v, stage_v, acc, dsem):
    c = lax.axis_index("c")
    s = lax.axis_index("s")
    wid = c * NS + s
    for j in range(CHUNK // 16):
        ones_v[pl.ds(j * 16, 16)] = jnp.ones((16,), jnp.float32)
        zero_v[pl.ds(j * 16, 16)] = jnp.zeros((16,), jnp.float32)
    for j in range(RPTD // CHUNK):
        pltpu.sync_copy(zero_v, acc.at[pl.ds(s * RPTD + j * CHUNK, CHUNK)])
    rem = RPTD % CHUNK
    if rem:
        pltpu.sync_copy(zero_v.at[pl.ds(0, rem)],
                        acc.at[pl.ds(s * RPTD + (RPTD // CHUNK) * CHUNK, rem)])
    pltpu.sync_copy(src_hbm.at[pl.ds(wid * NCHUNK, NCHUNK)], sidx)
    plsc.subcore_barrier()

    LAG = 8

    def body(k, carry):
        pltpu.async_copy(ones_v, acc.at[sidx.at[k]], dsem, add=True)

        @pl.when(k >= LAG)
        def _():
            pltpu.make_async_copy(ones_v, acc.at[sidx.at[k - LAG]],
                                  dsem).wait()
        return carry

    lax.fori_loop(0, NCHUNK, body, 0)

    def drain(k, carry):
        pltpu.make_async_copy(ones_v, acc.at[sidx.at[k]], dsem).wait()
        return carry

    lax.fori_loop(NCHUNK - LAG, NCHUNK, drain, 0)
    plsc.subcore_barrier()
    pltpu.sync_copy(acc.at[pl.ds(s * RPTD, RPTD)], stage_v)
    pltpu.sync_copy(stage_v, out_hbm.at[pl.ds(c * NDEG + s * RPTD, RPTD)])


def _sc_layer_body(table_hbm, src_hbm, dst_hbm, zeros_hbm, out_hbm,
                   didx, sidx, rows, acc, *sems):
    c = lax.axis_index("c")
    s = lax.axis_index("s")
    wid = c * NS + s
    gsem = sems[:NBUF]
    ssem = sems[NBUF:NBUF + 2]
    zsem = sems[NBUF + 2]
    zinit = pltpu.async_copy(zeros_hbm.at[pl.ds(s * RPT, RPT)],
                             acc.at[pl.ds(s * RPT, RPT)], zsem)
    pltpu.sync_copy(dst_hbm.at[pl.ds(wid * NCHUNK, NCHUNK)], didx)

    def sidx_src(g):
        return src_hbm.at[pl.ds(wid * NCHUNK + g * NBUF, NBUF)]

    for t in range(2):
        pltpu.async_copy(sidx_src(t), sidx.at[t], ssem[t])
    for b in range(NBUF):
        pltpu.async_copy(table_hbm.at[didx.at[b]], rows.at[b], gsem[b])
    zinit.wait()
    plsc.subcore_barrier()

    def one_group(g, slot):
        k0 = g * NBUF
        pltpu.make_async_copy(sidx_src(g), sidx.at[slot], ssem[slot]).wait()
        for b in range(NBUF):
            k = k0 + b
            pltpu.make_async_copy(table_hbm.at[didx.at[b]], rows.at[b],
                                  gsem[b]).wait()
            pltpu.sync_copy(rows.at[b], acc.at[sidx.at[slot, b]], add=True)

            @pl.when(g < GROUPS - 1)
            def _():
                pltpu.async_copy(table_hbm.at[didx.at[k + NBUF]], rows.at[b],
                                 gsem[b])

        @pl.when(g < GROUPS - 2)
        def _():
            pltpu.async_copy(sidx_src(g + 2), sidx.at[slot], ssem[slot])

    def super_group(h, carry):
        one_group(2 * h, 0)
        one_group(2 * h + 1, 1)
        return carry

    lax.fori_loop(0, GROUPS // 2, super_group, 0)
    plsc.subcore_barrier()
    pltpu.sync_copy(acc.at[pl.ds(s * RPT, RPT)], out_hbm.at[c, pl.ds(s * RPT, RPT)])


def _dinv_block(deg0_ref, deg1_ref):
    deg = 1.0 + deg0_ref[...] + deg1_ref[...]
    return lax.rsqrt(deg)


def _tc_a_body(x_ref, w1_ref, deg0_ref, deg1_ref, out_ref):
    p = lax.dot_general(x_ref[...], w1_ref[...], (((1,), (1,)), ((), ())),
                        preferred_element_type=jnp.float32)
    out_ref[...] = p * _dinv_block(deg0_ref, deg1_ref)


def _tc_b_body(zp_ref, yprev_ref, deg0_ref, deg1_ref, w2_ref, out_ref):
    dinv = _dinv_block(deg0_ref, deg1_ref)
    z = zp_ref[0] + zp_ref[1] + yprev_ref[...]
    h = jnp.maximum(dinv * z, 0.0)
    out_ref[...] = lax.dot_general(dinv * h, w2_ref[...], (((1,), (1,)), ((), ())),
                                   preferred_element_type=jnp.float32)


def _tc_c_body(zp_ref, yprev_ref, deg0_ref, deg1_ref, fcw_ref, fcb_ref, out_ref,
               acc_ref):
    i = pl.program_id(0)

    @pl.when(i == 0)
    def _():
        acc_ref[...] = jnp.zeros_like(acc_ref)

    dinv = _dinv_block(deg0_ref, deg1_ref)
    z = zp_ref[0] + zp_ref[1] + yprev_ref[...]
    h = jnp.maximum(dinv * z, 0.0)
    acc_ref[...] += jnp.sum(h, axis=0, keepdims=True)

    @pl.when(i == pl.num_programs(0) - 1)
    def _():
        m = acc_ref[...] * (1.0 / N)
        out_ref[...] = lax.dot_general(m, fcw_ref[...], (((1,), (1,)), ((), ())),
                                       preferred_element_type=jnp.float32) + fcb_ref[...]


_RB = 1000


def _row_specs():
    deg_spec = pl.BlockSpec((_RB, 1), lambda i: (i, 0))
    row_spec = pl.BlockSpec((_RB, D), lambda i: (i, 0))
    z_spec = pl.BlockSpec((NC, _RB, D), lambda i: (0, i, 0))
    w_spec = pl.BlockSpec((D, D), lambda i: (0, 0))
    return deg_spec, row_spec, z_spec, w_spec


def _tc_a(x, w1, deg0, deg1):
    deg_spec, row_spec, _, w_spec = _row_specs()
    return pl.pallas_call(
        _tc_a_body,
        grid=(N // _RB,),
        in_specs=[row_spec, w_spec, deg_spec, deg_spec],
        out_specs=row_spec,
        out_shape=jax.ShapeDtypeStruct((N, D), jnp.float32),
    )(x, w1, deg0, deg1)


def _tc_b(zp, yprev, deg0, deg1, w2):
    deg_spec, row_spec, z_spec, w_spec = _row_specs()
    return pl.pallas_call(
        _tc_b_body,
        grid=(N // _RB,),
        in_specs=[z_spec, row_spec, deg_spec, deg_spec, w_spec],
        out_specs=row_spec,
        out_shape=jax.ShapeDtypeStruct((N, D), jnp.float32),
    )(zp, yprev, deg0, deg1, w2)


def _tc_c(zp, yprev, deg0, deg1, fcw, fcb):
    deg_spec, row_spec, z_spec, _ = _row_specs()
    fcw_spec = pl.BlockSpec((D_OUT * D_OUT, D), lambda i: (0, 0))
    fcb_spec = pl.BlockSpec((1, D_OUT * D_OUT), lambda i: (0, 0))
    out_spec = pl.BlockSpec((1, D_OUT * D_OUT), lambda i: (0, 0))
    return pl.pallas_call(
        _tc_c_body,
        grid=(N // _RB,),
        in_specs=[z_spec, row_spec, deg_spec, deg_spec, fcw_spec, fcb_spec],
        out_specs=out_spec,
        out_shape=jax.ShapeDtypeStruct((1, D_OUT * D_OUT), jnp.float32),
        scratch_shapes=[pltpu.VMEM((1, D), jnp.float32)],
    )(zp, yprev, deg0, deg1, fcw, fcb)


def kernel(x, edge_index, W1, W2, fcW, fcb):
    src = edge_index[0].astype(jnp.int32)
    dst = edge_index[1].astype(jnp.int32)
    npad = EPAD - E
    pad_i = jnp.arange(npad, dtype=jnp.int32)
    src_p = jnp.concatenate([src, N + pad_i % (NROWS - N)]).reshape(-1, CHUNK)
    dst_p = jnp.concatenate([dst, pad_i % N]).reshape(-1, CHUNK)

    zerosD = jnp.zeros((NROWS, D), jnp.float32)

    sc_deg, sc_layer = _sc_kernels()
    degp = sc_deg(src_p)
    deg0 = degp[0:N].reshape(N, 1)
    deg1 = degp[NDEG:NDEG + N].reshape(N, 1)
    y0 = _tc_a(x, W1, deg0, deg1)
    z1 = sc_layer(y0, src_p, dst_p, zerosD)
    y1 = _tc_b(z1, y0, deg0, deg1, W2)
    z2 = sc_layer(y1, src_p, dst_p, zerosD)
    a_flat = _tc_c(z2, y1, deg0, deg1, fcW, fcb.reshape(1, -1))
    return a_flat.reshape(D_OUT, D_OUT)
